# baseline (speedup 1.0000x reference)
import functools

import jax
import jax.numpy as jnp
from jax import lax
from jax.experimental import pallas as pl
from jax.experimental.pallas import tpu as pltpu
from jax.experimental.pallas import tpu_sc as plsc

N = 50000
D = 128
E = 250000

NC = 2
NS = 16
L = 16

CHUNK = 128
CPT = 128
EPT = CPT * CHUNK
E_PAD = NS * EPT
QC = 32
NRING = 4
DUMMY = 50040
NT = 50048
NA = 51200
STRIPE = NA // NS
HSTRIPE = NT // NS
QD = 32

@functools.lru_cache(maxsize=None)
def _sc_mesh():
    return plsc.VectorSubcoreMesh(core_axis_name="c", subcore_axis_name="s",
                                  num_cores=NC, num_subcores=NS)


def _deg_body(edges_hbm, ones_hbm, zeros_hbm, deg_hbm,
              idx_v, ones_v, sem, acc_sh):
    c = lax.axis_index("c")
    s = lax.axis_index("s")
    pltpu.sync_copy(ones_hbm, ones_v)
    for j in range(2):
        a = 2 * c + j
        pltpu.sync_copy(zeros_hbm, acc_sh.at[pl.ds(s * HSTRIPE, HSTRIPE)])
        plsc.subcore_barrier()

        def step(k, carry):
            base = a * E_PAD + s * EPT + k * CHUNK
            pltpu.sync_copy(edges_hbm.at[pl.ds(base, CHUNK)], idx_v)
            pltpu.sync_copy(ones_v, acc_sh.at[idx_v], add=True)
            return carry

        lax.fori_loop(0, CPT, step, 0)
        plsc.subcore_barrier()
        pltpu.sync_copy(
            acc_sh.at[pl.ds(s * HSTRIPE, HSTRIPE)],
            deg_hbm.at[pl.ds(a * NT + s * HSTRIPE, HSTRIPE)])
        plsc.subcore_barrier()


@functools.lru_cache(maxsize=None)
def _deg_kernel():
    return pl.kernel(
        _deg_body,
        out_type=jax.ShapeDtypeStruct((4 * NT, L), jnp.float32),
        mesh=_sc_mesh(),
        scratch_types=[
            pltpu.VMEM((CHUNK,), jnp.int32),
            pltpu.VMEM((CHUNK, L), jnp.float32),
            pltpu.SemaphoreType.DMA,
            pltpu.VMEM_SHARED((NT, L), jnp.float32),
        ],
        compiler_params=pltpu.CompilerParams(use_tc_tiling_on_sc=False),
    )


def _agg_body(h0_hbm, h1_hbm, s0_hbm, d0_hbm, s1_hbm, d1_hbm, zeros_hbm,
              agg0_hbm, agg1_hbm,
              sblk, dblk, rows, gsem, acc_sh):
    c = lax.axis_index("c")
    s = lax.axis_index("s")
    NB = CPT // QC
    for r, (h_hbm, src_hbm, dst_hbm, agg_hbm) in enumerate(
            ((h0_hbm, s0_hbm, d0_hbm, agg0_hbm),
             (h1_hbm, s1_hbm, d1_hbm, agg1_hbm))):
        for j in range(2):
            q = 2 * c + j
            qoff = q * NT
            pltpu.sync_copy(zeros_hbm, acc_sh.at[pl.ds(s * STRIPE, STRIPE)])
            plsc.subcore_barrier()

            for blk in range(NB):
                pltpu.sync_copy(
                    src_hbm.at[pl.ds(s * EPT + blk * QC * CHUNK,
                                     QC * CHUNK)], sblk)
                pltpu.sync_copy(
                    dst_hbm.at[pl.ds(s * CPT + blk * QC, QC)], dblk)

                def adjust(i, carry):
                    sblk[pl.ds(i * L, L)] = sblk[pl.ds(i * L, L)] + qoff
                    return carry

                lax.fori_loop(0, QC * CHUNK // L, adjust, 0)

                def gather(k, p):
                    return pltpu.async_copy(
                        h_hbm.at[sblk.at[pl.ds(k * CHUNK, CHUNK)]],
                        rows[p], gsem[p])

                def wait_gather(p):
                    pltpu.make_async_copy(
                        h_hbm.at[sblk.at[pl.ds(0, CHUNK)]], rows[p],
                        gsem[p]).wait()

                gather(0, 0)
                gather(1, 1)

                def step(k2, carry):
                    k = 2 * k2
                    wait_gather(0)
                    pltpu.sync_copy(rows[0], acc_sh.at[dblk.at[k]],
                                    add=True)
                    gather(k + 2, 0)
                    wait_gather(1)
                    pltpu.sync_copy(rows[1], acc_sh.at[dblk.at[k + 1]],
                                    add=True)
                    gather(k + 3, 1)
                    return carry

                lax.fori_loop(0, QC // 2 - 1, step, 0)
                k = QC - 2
                wait_gather(0)
                pltpu.sync_copy(rows[0], acc_sh.at[dblk.at[k]], add=True)
                wait_gather(1)
                pltpu.sync_copy(rows[1], acc_sh.at[dblk.at[k + 1]],
                                add=True)

            plsc.subcore_barrier()
            pltpu.sync_copy(
                acc_sh.at[pl.ds(s * STRIPE, STRIPE)],
                agg_hbm.at[pl.ds(q * NA + s * STRIPE, STRIPE)])
            plsc.subcore_barrier()


@functools.lru_cache(maxsize=None)
def _agg_kernel():
    return pl.kernel(
        _agg_body,
        out_type=(jax.ShapeDtypeStruct((4 * NA, QD), jnp.float32),
                  jax.ShapeDtypeStruct((4 * NA, QD), jnp.float32)),
        mesh=_sc_mesh(),
        scratch_types=[
            pltpu.VMEM((QC * CHUNK,), jnp.int32),
            pltpu.VMEM((QC, CHUNK), jnp.int32),
            [pltpu.VMEM((CHUNK, QD), jnp.float32) for _ in range(2)],
            [pltpu.SemaphoreType.DMA for _ in range(2)],
            pltpu.VMEM_SHARED((NA, QD), jnp.float32),
        ],
        compiler_params=pltpu.CompilerParams(use_tc_tiling_on_sc=False),
    )


def _norm(deg):
    return lax.rsqrt(jnp.where(deg > 0.0, deg, 1.0))


def _mm(a, w):
    return lax.dot_general(a, w, (((1,), (0,)), ((), ())),
                           preferred_element_type=jnp.float32,
                           precision=lax.Precision.HIGHEST)


def _tc_a_body(x_ref, dS0_ref, dS1_ref, w0_ref, w1_ref, h0_ref, h1_ref):
    x = x_ref[...]
    h0 = _mm(x * _norm(dS0_ref[...]), w0_ref[...])
    h1 = _mm(x * _norm(dS1_ref[...]), w1_ref[...])
    for q in range(4):
        h0_ref[q] = h0[:, q * QD:(q + 1) * QD]
        h1_ref[q] = h1[:, q * QD:(q + 1) * QD]


def _tc_b_body(a0_ref, a1_ref, dD0_ref, dD1_ref, b0_ref, b1_ref,
               dS0_ref, dS1_ref, w0_ref, w1_ref, h0_ref, h1_ref):
    a0 = jnp.concatenate([a0_ref[q] for q in range(4)], axis=1)
    a1 = jnp.concatenate([a1_ref[q] for q in range(4)], axis=1)
    t = (a0 * _norm(dD0_ref[...]) + b0_ref[...]
         + a1 * _norm(dD1_ref[...]) + b1_ref[...])
    t = jnp.maximum(t, 0.0)
    h0 = _mm(t * _norm(dS0_ref[...]), w0_ref[...])
    h1 = _mm(t * _norm(dS1_ref[...]), w1_ref[...])
    for q in range(4):
        h0_ref[q] = h0[:, q * QD:(q + 1) * QD]
        h1_ref[q] = h1[:, q * QD:(q + 1) * QD]


def _tc_c_body(a0_ref, a1_ref, dD0_ref, dD1_ref, b0_ref, b1_ref, o_ref):
    a0 = jnp.concatenate([a0_ref[q] for q in range(4)], axis=1)
    a1 = jnp.concatenate([a1_ref[q] for q in range(4)], axis=1)
    t = (a0 * _norm(dD0_ref[...]) + b0_ref[...]
         + a1 * _norm(dD1_ref[...]) + b1_ref[...])
    o_ref[...] = jnp.maximum(t, 0.0)


TN = 1000
_GRID = N // TN

_col = pl.BlockSpec((TN, 1), lambda i: (i, 0))
_row = pl.BlockSpec((1, D), lambda i: (0, 0))
_wsp = pl.BlockSpec((D, D), lambda i: (0, 0))
_xsp = pl.BlockSpec((TN, D), lambda i: (i, 0))
_qsp = pl.BlockSpec((4, TN, QD), lambda i: (0, i, 0))
_asp = pl.BlockSpec((4, TN, QD), lambda i: (0, i, 0))

_h_shape = jax.ShapeDtypeStruct((4, NT, QD), jnp.float32)

_tc_a = pl.pallas_call(
    _tc_a_body, grid=(_GRID,),
    in_specs=[_xsp, _col, _col, _wsp, _wsp],
    out_specs=[_qsp, _qsp],
    out_shape=[_h_shape, _h_shape],
)

_tc_b = pl.pallas_call(
    _tc_b_body, grid=(_GRID,),
    in_specs=[_asp, _asp, _col, _col, _row, _row, _col, _col, _wsp, _wsp],
    out_specs=[_qsp, _qsp],
    out_shape=[_h_shape, _h_shape],
)

_tc_c = pl.pallas_call(
    _tc_c_body, grid=(_GRID,),
    in_specs=[_asp, _asp, _col, _col, _row, _row],
    out_specs=_xsp,
    out_shape=jax.ShapeDtypeStruct((N, D), jnp.float32),
)


def kernel(x, edge_index_r0, edge_index_r1,
           W1_0, b1_0, W1_1, b1_1, W2_0, b2_0, W2_1, b2_1):
    pad = jnp.full((E_PAD - E,), DUMMY, jnp.int32)
    s0 = jnp.concatenate([edge_index_r0[0].astype(jnp.int32), pad])
    d0 = jnp.concatenate([edge_index_r0[1].astype(jnp.int32), pad])
    s1 = jnp.concatenate([edge_index_r1[0].astype(jnp.int32), pad])
    d1 = jnp.concatenate([edge_index_r1[1].astype(jnp.int32), pad])
    edges_cat = jnp.concatenate([s0, d0, s1, d1])

    ones_rows = jnp.ones((CHUNK, L), jnp.float32)
    zeros_deg = jnp.zeros((HSTRIPE, L), jnp.float32)
    zeros_agg = jnp.zeros((STRIPE, QD), jnp.float32)

    hist = _deg_kernel()(edges_cat, ones_rows, zeros_deg)
    degS0 = hist[0 * NT:0 * NT + N, 0:1]
    degD0 = hist[1 * NT:1 * NT + N, 0:1]
    degS1 = hist[2 * NT:2 * NT + N, 0:1]
    degD1 = hist[3 * NT:3 * NT + N, 0:1]

    d0b = d0.reshape(NS * CPT, CHUNK)
    d1b = d1.reshape(NS * CPT, CHUNK)

    h0, h1 = _tc_a(x, degS0, degS1, W1_0, W1_1)
    a0, a1 = _agg_kernel()(h0.reshape(4 * NT, QD), h1.reshape(4 * NT, QD),
                           s0, d0b, s1, d1b, zeros_agg)
    h0, h1 = _tc_b(a0.reshape(4, NA, QD), a1.reshape(4, NA, QD),
                   degD0, degD1, b1_0.reshape(1, D),
                   b1_1.reshape(1, D), degS0, degS1, W2_0, W2_1)
    a0, a1 = _agg_kernel()(h0.reshape(4 * NT, QD), h1.reshape(4 * NT, QD),
                           s0, d0b, s1, d1b, zeros_agg)
    out = _tc_c(a0.reshape(4, NA, QD), a1.reshape(4, NA, QD),
                degD0, degD1, b2_0.reshape(1, D), b2_1.reshape(1, D))
    return out


# speedup vs baseline: 1.5278x; 1.5278x over previous
import functools

import jax
import jax.numpy as jnp
from jax import lax
from jax.experimental import pallas as pl
from jax.experimental.pallas import tpu as pltpu
from jax.experimental.pallas import tpu_sc as plsc

N = 50000
D = 128
E = 250000

NC = 2
NS = 16
L = 16

CHUNK = 128
CPT = 124
EPT = CPT * CHUNK
E_PAD = NS * EPT
QC = 62
DUMMY = 50040
NT = 50048
NA = 51200
STRIPE = NA // NS
HSTRIPE = NT // NS
QD = 32

@functools.lru_cache(maxsize=None)
def _sc_mesh():
    return plsc.VectorSubcoreMesh(core_axis_name="c", subcore_axis_name="s",
                                  num_cores=NC, num_subcores=NS)


def _deg_body(edges_hbm, ones_hbm, zeros_hbm, deg_hbm,
              idx_v, ones_v, sem, acc_sh):
    c = lax.axis_index("c")
    s = lax.axis_index("s")
    pltpu.sync_copy(ones_hbm, ones_v)
    for j in range(2):
        a = 2 * c + j
        pltpu.sync_copy(zeros_hbm, acc_sh.at[pl.ds(s * HSTRIPE, HSTRIPE)])
        plsc.subcore_barrier()

        def step(k, carry):
            base = a * E_PAD + s * EPT + k * CHUNK
            pltpu.sync_copy(edges_hbm.at[pl.ds(base, CHUNK)], idx_v)
            pltpu.sync_copy(ones_v, acc_sh.at[idx_v], add=True)
            return carry

        lax.fori_loop(0, CPT, step, 0)
        plsc.subcore_barrier()
        pltpu.sync_copy(
            acc_sh.at[pl.ds(s * HSTRIPE, HSTRIPE)],
            deg_hbm.at[pl.ds(a * NT + s * HSTRIPE, HSTRIPE)])
        plsc.subcore_barrier()


@functools.lru_cache(maxsize=None)
def _deg_kernel():
    return pl.kernel(
        _deg_body,
        out_type=jax.ShapeDtypeStruct((4 * NT, L), jnp.float32),
        mesh=_sc_mesh(),
        scratch_types=[
            pltpu.VMEM((CHUNK,), jnp.int32),
            pltpu.VMEM((CHUNK, L), jnp.float32),
            pltpu.SemaphoreType.DMA,
            pltpu.VMEM_SHARED((NT, L), jnp.float32),
        ],
        compiler_params=pltpu.CompilerParams(use_tc_tiling_on_sc=False),
    )


def _agg_body(h0_hbm, h1_hbm, s0_hbm, d0_hbm, s1_hbm, d1_hbm, zeros_hbm,
              agg0_hbm, agg1_hbm,
              sblk, dblk, rows, gsem, acc_sh):
    c = lax.axis_index("c")
    s = lax.axis_index("s")
    NB = CPT // QC
    for r, (h_hbm, src_hbm, dst_hbm, agg_hbm) in enumerate(
            ((h0_hbm, s0_hbm, d0_hbm, agg0_hbm),
             (h1_hbm, s1_hbm, d1_hbm, agg1_hbm))):
        for j in range(2):
            q = 2 * c + j
            qoff = q * NT
            pltpu.sync_copy(zeros_hbm, acc_sh.at[pl.ds(s * STRIPE, STRIPE)])
            plsc.subcore_barrier()

            for blk in range(NB):
                pltpu.sync_copy(
                    src_hbm.at[pl.ds(s * EPT + blk * QC * CHUNK,
                                     QC * CHUNK)], sblk)
                pltpu.sync_copy(
                    dst_hbm.at[pl.ds(s * CPT + blk * QC, QC)], dblk)

                def adjust(i, carry):
                    sblk[pl.ds(i * L, L)] = sblk[pl.ds(i * L, L)] + qoff
                    return carry

                lax.fori_loop(0, QC * CHUNK // L, adjust, 0)

                def gather(k, p):
                    return pltpu.async_copy(
                        h_hbm.at[sblk.at[pl.ds(k * CHUNK, CHUNK)]],
                        rows[p], gsem[p])

                def wait_gather(p):
                    pltpu.make_async_copy(
                        h_hbm.at[sblk.at[pl.ds(0, CHUNK)]], rows[p],
                        gsem[p]).wait()

                gather(0, 0)
                gather(1, 1)

                def step(k2, carry):
                    k = 2 * k2
                    wait_gather(0)
                    pltpu.sync_copy(rows[0], acc_sh.at[dblk.at[k]],
                                    add=True)
                    gather(k + 2, 0)
                    wait_gather(1)
                    pltpu.sync_copy(rows[1], acc_sh.at[dblk.at[k + 1]],
                                    add=True)
                    gather(k + 3, 1)
                    return carry

                lax.fori_loop(0, QC // 2 - 1, step, 0)
                k = QC - 2
                wait_gather(0)
                pltpu.sync_copy(rows[0], acc_sh.at[dblk.at[k]], add=True)
                wait_gather(1)
                pltpu.sync_copy(rows[1], acc_sh.at[dblk.at[k + 1]],
                                add=True)

            plsc.subcore_barrier()
            pltpu.sync_copy(
                acc_sh.at[pl.ds(s * STRIPE, STRIPE)],
                agg_hbm.at[pl.ds(q * NA + s * STRIPE, STRIPE)])
            plsc.subcore_barrier()


@functools.lru_cache(maxsize=None)
def _agg_kernel():
    return pl.kernel(
        _agg_body,
        out_type=(jax.ShapeDtypeStruct((4 * NA, QD), jnp.float32),
                  jax.ShapeDtypeStruct((4 * NA, QD), jnp.float32)),
        mesh=_sc_mesh(),
        scratch_types=[
            pltpu.VMEM((QC * CHUNK,), jnp.int32),
            pltpu.VMEM((QC, CHUNK), jnp.int32),
            [pltpu.VMEM((CHUNK, QD), jnp.float32) for _ in range(2)],
            [pltpu.SemaphoreType.DMA for _ in range(2)],
            pltpu.VMEM_SHARED((NA, QD), jnp.float32),
        ],
        compiler_params=pltpu.CompilerParams(use_tc_tiling_on_sc=False),
    )


def _norm(deg):
    return lax.rsqrt(jnp.where(deg > 0.0, deg, 1.0))


def _mm(a, w):
    return lax.dot_general(a, w, (((1,), (0,)), ((), ())),
                           preferred_element_type=jnp.float32,
                           precision=lax.Precision.HIGHEST)


def _tc_a_body(x_ref, dS0_ref, dS1_ref, w0_ref, w1_ref, h0_ref, h1_ref):
    x = x_ref[...]
    h0 = _mm(x * _norm(dS0_ref[...]), w0_ref[...])
    h1 = _mm(x * _norm(dS1_ref[...]), w1_ref[...])
    for q in range(4):
        h0_ref[q] = h0[:, q * QD:(q + 1) * QD]
        h1_ref[q] = h1[:, q * QD:(q + 1) * QD]


def _tc_b_body(a0_ref, a1_ref, dD0_ref, dD1_ref, b0_ref, b1_ref,
               dS0_ref, dS1_ref, w0_ref, w1_ref, h0_ref, h1_ref):
    a0 = jnp.concatenate([a0_ref[q] for q in range(4)], axis=1)
    a1 = jnp.concatenate([a1_ref[q] for q in range(4)], axis=1)
    t = (a0 * _norm(dD0_ref[...]) + b0_ref[...]
         + a1 * _norm(dD1_ref[...]) + b1_ref[...])
    t = jnp.maximum(t, 0.0)
    h0 = _mm(t * _norm(dS0_ref[...]), w0_ref[...])
    h1 = _mm(t * _norm(dS1_ref[...]), w1_ref[...])
    for q in range(4):
        h0_ref[q] = h0[:, q * QD:(q + 1) * QD]
        h1_ref[q] = h1[:, q * QD:(q + 1) * QD]


def _tc_c_body(a0_ref, a1_ref, dD0_ref, dD1_ref, b0_ref, b1_ref, o_ref):
    a0 = jnp.concatenate([a0_ref[q] for q in range(4)], axis=1)
    a1 = jnp.concatenate([a1_ref[q] for q in range(4)], axis=1)
    t = (a0 * _norm(dD0_ref[...]) + b0_ref[...]
         + a1 * _norm(dD1_ref[...]) + b1_ref[...])
    o_ref[...] = jnp.maximum(t, 0.0)


TN = 1000
_GRID = N // TN

_col = pl.BlockSpec((TN, 1), lambda i: (i, 0))
_row = pl.BlockSpec((1, D), lambda i: (0, 0))
_wsp = pl.BlockSpec((D, D), lambda i: (0, 0))
_xsp = pl.BlockSpec((TN, D), lambda i: (i, 0))
_qsp = pl.BlockSpec((4, TN, QD), lambda i: (0, i, 0))
_asp = pl.BlockSpec((4, TN, QD), lambda i: (0, i, 0))

_h_shape = jax.ShapeDtypeStruct((4, NT, QD), jnp.float32)

_tc_a = pl.pallas_call(
    _tc_a_body, grid=(_GRID,),
    in_specs=[_xsp, _col, _col, _wsp, _wsp],
    out_specs=[_qsp, _qsp],
    out_shape=[_h_shape, _h_shape],
)

_tc_b = pl.pallas_call(
    _tc_b_body, grid=(_GRID,),
    in_specs=[_asp, _asp, _col, _col, _row, _row, _col, _col, _wsp, _wsp],
    out_specs=[_qsp, _qsp],
    out_shape=[_h_shape, _h_shape],
)

_tc_c = pl.pallas_call(
    _tc_c_body, grid=(_GRID,),
    in_specs=[_asp, _asp, _col, _col, _row, _row],
    out_specs=_xsp,
    out_shape=jax.ShapeDtypeStruct((N, D), jnp.float32),
)


def kernel(x, edge_index_r0, edge_index_r1,
           W1_0, b1_0, W1_1, b1_1, W2_0, b2_0, W2_1, b2_1):
    pad = N + (jnp.arange(E_PAD - E, dtype=jnp.int32) % 48)
    s0 = jnp.concatenate([edge_index_r0[0].astype(jnp.int32), pad])
    d0 = jnp.concatenate([edge_index_r0[1].astype(jnp.int32), pad])
    s1 = jnp.concatenate([edge_index_r1[0].astype(jnp.int32), pad])
    d1 = jnp.concatenate([edge_index_r1[1].astype(jnp.int32), pad])
    edges_cat = jnp.concatenate([s0, d0, s1, d1])

    ones_rows = jnp.ones((CHUNK, L), jnp.float32)
    zeros_deg = jnp.zeros((HSTRIPE, L), jnp.float32)
    zeros_agg = jnp.zeros((STRIPE, QD), jnp.float32)

    hist = _deg_kernel()(edges_cat, ones_rows, zeros_deg)
    degS0 = hist[0 * NT:0 * NT + N, 0:1]
    degD0 = hist[1 * NT:1 * NT + N, 0:1]
    degS1 = hist[2 * NT:2 * NT + N, 0:1]
    degD1 = hist[3 * NT:3 * NT + N, 0:1]

    d0b = d0.reshape(NS * CPT, CHUNK)
    d1b = d1.reshape(NS * CPT, CHUNK)

    h0, h1 = _tc_a(x, degS0, degS1, W1_0, W1_1)
    a0, a1 = _agg_kernel()(h0.reshape(4 * NT, QD), h1.reshape(4 * NT, QD),
                           s0, d0b, s1, d1b, zeros_agg)
    h0, h1 = _tc_b(a0.reshape(4, NA, QD), a1.reshape(4, NA, QD),
                   degD0, degD1, b1_0.reshape(1, D),
                   b1_1.reshape(1, D), degS0, degS1, W2_0, W2_1)
    a0, a1 = _agg_kernel()(h0.reshape(4 * NT, QD), h1.reshape(4 * NT, QD),
                           s0, d0b, s1, d1b, zeros_agg)
    out = _tc_c(a0.reshape(4, NA, QD), a1.reshape(4, NA, QD),
                degD0, degD1, b2_0.reshape(1, D), b2_1.reshape(1, D))
    return out
